# baseline (speedup 1.0000x reference)
import jax
import jax.numpy as jnp
from jax.experimental import pallas as pl

NUM_EXPERTS = 16
HIDDEN = 64
FLAT = NUM_EXPERTS * HIDDEN
GPAD = 128


def _moe_body(x_ref, wcat_ref, bcat_ref, gw2t_ref, gb2_ref,
              w2_ref, eb2_ref, emat_ref, out_ref):
    xb = x_ref[...]
    f = xb.shape[0]
    xb16 = xb.astype(jnp.bfloat16)

    hall = jnp.maximum(
        jnp.dot(xb16, wcat_ref[...], preferred_element_type=jnp.float32)
        + bcat_ref[...], 0.0)
    h = hall[:, :FLAT]
    g1 = hall[:, FLAT:FLAT + HIDDEN]

    logits = (jnp.dot(g1, gw2t_ref[...], preferred_element_type=jnp.float32)
              + gb2_ref[...])
    m = jnp.max(logits, axis=1, keepdims=True)
    el = jnp.exp(logits - m)
    z = jnp.sum(el, axis=1, keepdims=True)

    m1 = jnp.max(el, axis=1, keepdims=True)
    el2 = jnp.where(el == m1, -1.0, el)
    m2 = jnp.max(el2, axis=1, keepdims=True)
    w = jnp.where(el >= m2, el, 0.0) / z

    c = jax.lax.dot_general(w, h, (((0,), (0,)), ((), ())),
                            preferred_element_type=jnp.float32)
    s = jnp.sum(c * emat_ref[...], axis=0, keepdims=True)
    wsum = jnp.sum(w, axis=0, keepdims=True)

    out = (jnp.dot(s, w2_ref[...], preferred_element_type=jnp.float32)
           + jnp.dot(wsum, eb2_ref[...], preferred_element_type=jnp.float32))
    out_ref[...] = (out * (1.0 / f))[None]


def kernel(x, gw1, gb1, gw2, gb2, ew1, eb1, ew2, eb2):
    B, F, D = x.shape
    E, H, _ = ew1.shape
    O = ew2.shape[1]

    xf = x.reshape(B * F, D)
    w1t = ew1.reshape(E * H, D).T
    gpad = jnp.zeros((D, GPAD - H), dtype=x.dtype)
    wcat = jnp.concatenate([w1t, gw1.T, gpad], axis=1).astype(jnp.bfloat16)
    bcat = jnp.concatenate(
        [eb1.reshape(E * H), gb1, jnp.zeros((GPAD - H,), x.dtype)]
    ).reshape(1, E * H + GPAD)
    gw2t = gw2.T
    gb2r = gb2.reshape(1, E)
    w2 = ew2.transpose(0, 2, 1).reshape(E * H, O)
    emat = jnp.kron(jnp.eye(E, dtype=x.dtype), jnp.ones((1, H), dtype=x.dtype))

    full = lambda *shape: pl.BlockSpec(shape, lambda b: (0,) * len(shape))
    out = pl.pallas_call(
        _moe_body,
        grid=(B,),
        in_specs=[
            pl.BlockSpec((F, D), lambda b: (b, 0)),
            full(D, E * H + GPAD), full(1, E * H + GPAD),
            full(H, E), full(1, E),
            full(E * H, O), full(E, O),
            full(E, E * H),
        ],
        out_specs=pl.BlockSpec((1, 1, O), lambda b: (b, 0, 0)),
        out_shape=jax.ShapeDtypeStruct((B, 1, O), x.dtype),
    )(xf, wcat, bcat, gw2t, gb2r, w2, eb2, emat)
    return out.reshape(B, 1, 1, O)


# speedup vs baseline: 1.0851x; 1.0851x over previous
import jax
import jax.numpy as jnp
from jax.experimental import pallas as pl

NUM_EXPERTS = 16
HIDDEN = 64
FLAT = NUM_EXPERTS * HIDDEN


def _moe_body(x_ref, gw1t_ref, gb1_ref, gw2t_ref, gb2_ref,
              w1t_ref, b1_ref, w2_ref, eb2_ref, emat_ref, out_ref):
    xb = x_ref[...]
    f = xb.shape[0]
    xb16 = xb.astype(jnp.bfloat16)

    g1 = jnp.maximum(
        jnp.dot(xb16, gw1t_ref[...], preferred_element_type=jnp.float32)
        + gb1_ref[...], 0.0)
    logits = (jnp.dot(g1, gw2t_ref[...], preferred_element_type=jnp.float32)
              + gb2_ref[...])
    m = jnp.max(logits, axis=1, keepdims=True)
    el = jnp.exp(logits - m)
    z = jnp.sum(el, axis=1, keepdims=True)

    m1 = jnp.max(el, axis=1, keepdims=True)
    el2 = jnp.where(el == m1, -1.0, el)
    m2 = jnp.max(el2, axis=1, keepdims=True)
    w = jnp.where(el >= m2, el, 0.0) / z

    h = jnp.maximum(
        jnp.dot(xb16, w1t_ref[...], preferred_element_type=jnp.float32)
        + b1_ref[...], 0.0)

    c = jax.lax.dot_general(w, h, (((0,), (0,)), ((), ())),
                            preferred_element_type=jnp.float32)
    s = jnp.sum(c * emat_ref[...], axis=0, keepdims=True)
    wsum = jnp.sum(w, axis=0, keepdims=True)

    out = (jnp.dot(s, w2_ref[...], preferred_element_type=jnp.float32)
           + jnp.dot(wsum, eb2_ref[...], preferred_element_type=jnp.float32))
    out_ref[...] = (out * (1.0 / f))[None]


def kernel(x, gw1, gb1, gw2, gb2, ew1, eb1, ew2, eb2):
    B, F, D = x.shape
    E, H, _ = ew1.shape
    O = ew2.shape[1]

    xf = x.reshape(B * F, D)
    gw1t = gw1.T.astype(jnp.bfloat16)
    gw2t = gw2.T
    gb1r = gb1.reshape(1, H)
    gb2r = gb2.reshape(1, E)
    w1t = ew1.reshape(E * H, D).T.astype(jnp.bfloat16)
    b1r = eb1.reshape(1, E * H)
    w2 = ew2.transpose(0, 2, 1).reshape(E * H, O)
    emat = jnp.kron(jnp.eye(E, dtype=x.dtype), jnp.ones((1, H), dtype=x.dtype))

    full = lambda *shape: pl.BlockSpec(shape, lambda b: (0,) * len(shape))
    out = pl.pallas_call(
        _moe_body,
        grid=(B,),
        in_specs=[
            pl.BlockSpec((F, D), lambda b: (b, 0)),
            full(D, H), full(1, H), full(H, E), full(1, E),
            full(D, E * H), full(1, E * H), full(E * H, O), full(E, O),
            full(E, E * H),
        ],
        out_specs=pl.BlockSpec((1, 1, O), lambda b: (b, 0, 0)),
        out_shape=jax.ShapeDtypeStruct((B, 1, O), x.dtype),
    )(xf, gw1t, gb1r, gw2t, gb2r, w1t, b1r, w2, eb2, emat)
    return out.reshape(B, 1, 1, O)
